# baseline (speedup 1.0000x reference)
import functools

import jax
import jax.numpy as jnp
from jax import lax
from jax.experimental import pallas as pl
from jax.experimental.pallas import tpu as pltpu
from jax.experimental.pallas import tpu_sc as plsc

N_NODES = 10000
D_FEAT = 256
S_NEIGH = 16
D_HIDDEN = 256
K_ASP = 8
D_OUT = 128
NUM_NEGS = 20
BATCH = 256
ALPHA = 1.0
BETA = 1.0
GAMMA = 0.1

M_PAD = 24
ROWS = BATCH * M_PAD
HALF_ROWS = ROWS // 4
NUM_CORES = 2
NUM_SUBCORES = 16
NW = NUM_CORES * NUM_SUBCORES
RPW = HALF_ROWS // NW
CH = 8
NCH = RPW // CH
LANES = 16

BB = 32
TC_ROWS = BB * M_PAD
GRID = (BATCH // 4) // BB


def _sc_gather_body(ids_hbm, ntp_hbm, feat_hbm, out_hbm,
                    idx_v, nidwa_v, nidwb_v, fidx_v,
                    selfb0_v, selfb1_v,
                    nrows0_v, nrows1_v, outc0_v, outc1_v, outc2_v,
                    sem_n, sem_f0, sem_f1, sem_o0, sem_o1, sem_o2):
    wid = lax.axis_index("s") * NUM_CORES + lax.axis_index("c")
    base = wid * RPW

    selfb = (selfb0_v, selfb1_v)
    nrows = (nrows0_v, nrows1_v)
    outc = (outc0_v, outc1_v, outc2_v)
    sem_f = (sem_f0, sem_f1)
    sem_o = (sem_o0, sem_o1, sem_o2)

    pltpu.sync_copy(ids_hbm.at[pl.ds(base, RPW)], idx_v)
    half = RPW // 2
    hn0 = pltpu.async_copy(ntp_hbm.at[idx_v.at[pl.ds(0, half)]],
                           nidwa_v, sem_n)
    hn1 = pltpu.async_copy(ntp_hbm.at[idx_v.at[pl.ds(half, half)]],
                           nidwb_v, sem_n)
    hn0.wait()
    hn1.wait()
    for r in range(RPW):
        src = nidwa_v if r < half else nidwb_v
        fidx_v[r // CH, pl.ds((r % CH) * LANES, LANES)] = src[r % half, :LANES]

    def issue_feat(c):
        return (
            pltpu.async_copy(feat_hbm.at[idx_v.at[pl.ds(c * CH, CH)]],
                             selfb[c % 2], sem_f[c % 2]),
            pltpu.async_copy(feat_hbm.at[fidx_v.at[c]], nrows[c % 2],
                             sem_f[c % 2]),
        )

    NG = D_FEAT // LANES

    def reduce(c):
        b2 = c % 2
        b3 = c % 3

        def slot(j, carry2):
            def col(g, carry3):
                sl = pl.ds(g * LANES, LANES)
                acc = nrows[b2][j * S_NEIGH, sl]
                for s in range(1, S_NEIGH):
                    acc = acc + nrows[b2][j * S_NEIGH + s, sl]
                outc[b3][j, pl.ds(D_FEAT + g * LANES, LANES)] = acc
                outc[b3][j, sl] = selfb[b2][j, sl]
                return carry3
            return lax.fori_loop(0, NG, col, carry2)
        lax.fori_loop(0, CH, slot, 0)
        return pltpu.async_copy(
            outc[b3], out_hbm.at[pl.ds(base + c * CH, CH), :], sem_o[b3])

    hf = {}
    ho = {}
    hf[0] = issue_feat(0)
    hf[1] = issue_feat(1)
    for c in range(NCH):
        for h in hf[c]:
            h.wait()
        if c >= 3:
            ho[c - 3].wait()
        ho[c] = reduce(c)
        if c + 2 < NCH:
            hf[c + 2] = issue_feat(c + 2)
    ho[NCH - 2].wait()
    ho[NCH - 1].wait()


def _sc_gather(ids_half, ntp, features):
    mesh = plsc.VectorSubcoreMesh(core_axis_name="c", subcore_axis_name="s")
    kern = functools.partial(
        pl.kernel, mesh=mesh,
        out_type=jax.ShapeDtypeStruct((HALF_ROWS, 2 * D_FEAT), jnp.float32),
        scratch_types=[
            pltpu.VMEM((RPW,), jnp.int32),
            pltpu.VMEM((RPW // 2, 128), jnp.int32),
            pltpu.VMEM((RPW // 2, 128), jnp.int32),
            pltpu.VMEM((NCH, 128), jnp.int32),
            pltpu.VMEM((CH, D_FEAT), jnp.float32),
            pltpu.VMEM((CH, D_FEAT), jnp.float32),
            pltpu.VMEM((CH * S_NEIGH, D_FEAT), jnp.float32),
            pltpu.VMEM((CH * S_NEIGH, D_FEAT), jnp.float32),
            pltpu.VMEM((CH, 2 * D_FEAT), jnp.float32),
            pltpu.VMEM((CH, 2 * D_FEAT), jnp.float32),
            pltpu.VMEM((CH, 2 * D_FEAT), jnp.float32),
            pltpu.SemaphoreType.DMA,
            pltpu.SemaphoreType.DMA,
            pltpu.SemaphoreType.DMA,
            pltpu.SemaphoreType.DMA,
            pltpu.SemaphoreType.DMA,
            pltpu.SemaphoreType.DMA,
        ],
    )(_sc_gather_body)
    return kern(ids_half, ntp, features)


def _tc_loss_body(g_ref, w1_ref, wflat_ref, out_ref):
    i = pl.program_id(0)

    g = g_ref[...]
    h = jnp.dot(g[:, :D_FEAT], w1_ref[:D_FEAT, :],
                preferred_element_type=jnp.float32)
    h = h + jnp.dot(g[:, D_FEAT:], w1_ref[D_FEAT:, :],
                    preferred_element_type=jnp.float32) * (1.0 / S_NEIGH)
    h = jnp.maximum(h, 0.0)

    a = jnp.dot(h, wflat_ref[...],
                preferred_element_type=jnp.float32)

    rr = lax.broadcasted_iota(jnp.int32, (TC_ROWS, BB), 0)
    cc = lax.broadcasted_iota(jnp.int32, (TC_ROWS, BB), 1)
    psel = (rr == cc * M_PAD).astype(jnp.float32)
    pgrp = (rr // M_PAD == cc).astype(jnp.float32)

    def group_bcast(x):
        sel = jax.lax.dot_general(psel, x, (((0,), (0,)), ((), ())),
                                  preferred_element_type=jnp.float32)
        return jnp.dot(pgrp, sel, preferred_element_type=jnp.float32)

    abc = group_bcast(a)
    ls = jnp.sum((a * abc).reshape(BB, M_PAD, K_ASP * D_OUT), axis=-1)
    ls = ls * (1.0 / K_ASP)

    gmax = a[:, :D_OUT]
    for k in range(1, K_ASP):
        gmax = jnp.maximum(gmax, a[:, k * D_OUT:(k + 1) * D_OUT])
    gmaxbc = group_bcast(gmax)
    gs = jnp.sum((gmax * gmaxbc).reshape(BB, M_PAD, D_OUT), axis=-1)

    midx = lax.broadcasted_iota(jnp.int32, (BB, M_PAD), 1)
    valid = (midx >= 1) & (midx <= 1 + NUM_NEGS)

    def xent(scores):
        sm = jnp.where(valid, scores, -1e30)
        rmax = jnp.max(sm, axis=1, keepdims=True)
        se = jnp.sum(jnp.where(valid, jnp.exp(scores - rmax), 0.0),
                     axis=1, keepdims=True)
        row = jnp.log(se) + rmax - scores[:, 1:2]
        return jnp.sum(row) * (1.0 / BATCH)

    xent_g = xent(gs)
    xent_l = xent(ls)

    locs = [a[:, k * D_OUT:(k + 1) * D_OUT].reshape(BB, M_PAD, D_OUT)
            for k in range(K_ASP)]
    gram = [[None] * K_ASP for _ in range(K_ASP)]
    for k in range(K_ASP):
        for n in range(k, K_ASP):
            p = jnp.sum(locs[k] * locs[n], axis=-1)
            gram[k][n] = p
            gram[n][k] = p
    acc = jnp.zeros((BB, M_PAD), jnp.float32)
    for n in range(K_ASP):
        deno = gram[0][n]
        for k in range(1, K_ASP):
            deno = jnp.maximum(deno, gram[k][n])
        deno = jnp.where(deno == 0.0, 1.0, deno)
        inv = 1.0 / deno
        for k in range(K_ASP):
            tgt = 1.0 if k == n else 0.0
            acc = acc + jnp.abs(gram[k][n] * inv - tgt)
    w = jnp.where(midx == 0, 1.0 / BATCH,
                  jnp.where(valid, 1.0 / (BATCH * (1 + NUM_NEGS)), 0.0))
    constrain = jnp.sum(acc * w)

    contrib = ALPHA * xent_g + BETA * xent_l + GAMMA * constrain

    @pl.when(i == 0)
    def _():
        out_ref[...] = jnp.zeros((1, 1), jnp.float32)
    out_ref[...] = out_ref[...] + jnp.reshape(contrib, (1, 1))


def _tc_loss(G, W1, Wflat, interpret=False):
    return pl.pallas_call(
        _tc_loss_body,
        grid=(GRID,),
        in_specs=[
            pl.BlockSpec((TC_ROWS, 2 * D_FEAT), lambda i: (i, 0)),
            pl.BlockSpec((2 * D_FEAT, D_HIDDEN), lambda i: (0, 0)),
            pl.BlockSpec((D_HIDDEN, K_ASP * D_OUT), lambda i: (0, 0)),
        ],
        out_specs=pl.BlockSpec((1, 1), lambda i: (0, 0)),
        out_shape=jax.ShapeDtypeStruct((1, 1), jnp.float32),
        interpret=interpret,
    )(G, W1, Wflat)


def kernel(edges, negs, neigh_table, features, W1, W_asp):
    pad = jnp.zeros((BATCH, M_PAD - 2 - NUM_NEGS), jnp.int32)
    ids24 = jnp.concatenate(
        [edges[:, 0:1], edges[:, 1:2], negs, pad], axis=1)
    ids_flat = ids24.reshape(ROWS)
    Wflat = jnp.transpose(W_asp, (1, 0, 2)).reshape(D_HIDDEN, K_ASP * D_OUT)
    ntp = jnp.zeros((N_NODES, 128), jnp.int32).at[:, :S_NEIGH].set(neigh_table)
    gs = [_sc_gather(ids_flat[q * HALF_ROWS:(q + 1) * HALF_ROWS],
                     ntp, features) for q in range(4)]
    ls = [_tc_loss(g, W1, Wflat) for g in gs]
    return (ls[0] + ls[1] + ls[2] + ls[3])[0, 0]


# speedup vs baseline: 1.0309x; 1.0309x over previous
import functools

import jax
import jax.numpy as jnp
from jax import lax
from jax.experimental import pallas as pl
from jax.experimental.pallas import tpu as pltpu
from jax.experimental.pallas import tpu_sc as plsc

N_NODES = 10000
D_FEAT = 256
S_NEIGH = 16
D_HIDDEN = 256
K_ASP = 8
D_OUT = 128
NUM_NEGS = 20
BATCH = 256
ALPHA = 1.0
BETA = 1.0
GAMMA = 0.1

M_PAD = 24
ROWS = BATCH * M_PAD
HALF_ROWS = ROWS // 2
NUM_CORES = 2
NUM_SUBCORES = 16
NW = NUM_CORES * NUM_SUBCORES
RPW = HALF_ROWS // NW
CH = 8
NCH = RPW // CH
LANES = 16

BB = 32
TC_ROWS = BB * M_PAD
GRID = (BATCH // 2) // BB


def _sc_gather_body(ids_hbm, ntp_hbm, feat_hbm, out_hbm,
                    idx_v, nidwa_v, nidwb_v, fidx_v,
                    selfb0_v, selfb1_v,
                    nrows0_v, nrows1_v, outc0_v, outc1_v, outc2_v,
                    sem_n, sem_f0, sem_f1, sem_o0, sem_o1, sem_o2):
    wid = lax.axis_index("s") * NUM_CORES + lax.axis_index("c")
    base = wid * RPW

    selfb = (selfb0_v, selfb1_v)
    nrows = (nrows0_v, nrows1_v)
    outc = (outc0_v, outc1_v, outc2_v)
    sem_f = (sem_f0, sem_f1)
    sem_o = (sem_o0, sem_o1, sem_o2)

    pltpu.sync_copy(ids_hbm.at[pl.ds(base, RPW)], idx_v)
    half = RPW // 2
    hn0 = pltpu.async_copy(ntp_hbm.at[idx_v.at[pl.ds(0, half)]],
                           nidwa_v, sem_n)
    hn1 = pltpu.async_copy(ntp_hbm.at[idx_v.at[pl.ds(half, half)]],
                           nidwb_v, sem_n)
    hn0.wait()
    hn1.wait()
    for r in range(RPW):
        src = nidwa_v if r < half else nidwb_v
        fidx_v[r // CH, pl.ds((r % CH) * LANES, LANES)] = src[r % half, :LANES]

    def issue_feat(c):
        return (
            pltpu.async_copy(feat_hbm.at[idx_v.at[pl.ds(c * CH, CH)]],
                             selfb[c % 2], sem_f[c % 2]),
            pltpu.async_copy(feat_hbm.at[fidx_v.at[c]], nrows[c % 2],
                             sem_f[c % 2]),
        )

    NG = D_FEAT // LANES

    def reduce(c):
        b2 = c % 2
        b3 = c % 3

        def slot(j, carry2):
            def col(g, carry3):
                sl = pl.ds(g * LANES, LANES)
                acc = nrows[b2][j * S_NEIGH, sl]
                for s in range(1, S_NEIGH):
                    acc = acc + nrows[b2][j * S_NEIGH + s, sl]
                outc[b3][j, pl.ds(D_FEAT + g * LANES, LANES)] = acc
                outc[b3][j, sl] = selfb[b2][j, sl]
                return carry3
            return lax.fori_loop(0, NG, col, carry2)
        lax.fori_loop(0, CH, slot, 0)
        return pltpu.async_copy(
            outc[b3], out_hbm.at[pl.ds(base + c * CH, CH), :], sem_o[b3])

    hf = {}
    ho = {}
    hf[0] = issue_feat(0)
    hf[1] = issue_feat(1)
    for c in range(NCH):
        for h in hf[c]:
            h.wait()
        if c >= 3:
            ho[c - 3].wait()
        ho[c] = reduce(c)
        if c + 2 < NCH:
            hf[c + 2] = issue_feat(c + 2)
    ho[NCH - 2].wait()
    ho[NCH - 1].wait()


def _sc_gather(ids_half, ntp, features):
    mesh = plsc.VectorSubcoreMesh(core_axis_name="c", subcore_axis_name="s")
    kern = functools.partial(
        pl.kernel, mesh=mesh,
        out_type=jax.ShapeDtypeStruct((HALF_ROWS, 2 * D_FEAT), jnp.float32),
        scratch_types=[
            pltpu.VMEM((RPW,), jnp.int32),
            pltpu.VMEM((RPW // 2, 128), jnp.int32),
            pltpu.VMEM((RPW // 2, 128), jnp.int32),
            pltpu.VMEM((NCH, 128), jnp.int32),
            pltpu.VMEM((CH, D_FEAT), jnp.float32),
            pltpu.VMEM((CH, D_FEAT), jnp.float32),
            pltpu.VMEM((CH * S_NEIGH, D_FEAT), jnp.float32),
            pltpu.VMEM((CH * S_NEIGH, D_FEAT), jnp.float32),
            pltpu.VMEM((CH, 2 * D_FEAT), jnp.float32),
            pltpu.VMEM((CH, 2 * D_FEAT), jnp.float32),
            pltpu.VMEM((CH, 2 * D_FEAT), jnp.float32),
            pltpu.SemaphoreType.DMA,
            pltpu.SemaphoreType.DMA,
            pltpu.SemaphoreType.DMA,
            pltpu.SemaphoreType.DMA,
            pltpu.SemaphoreType.DMA,
            pltpu.SemaphoreType.DMA,
        ],
    )(_sc_gather_body)
    return kern(ids_half, ntp, features)


def _tc_loss_body(g_ref, w1_ref, wflat_ref, out_ref):
    i = pl.program_id(0)

    g = g_ref[...]
    h = jnp.dot(g[:, :D_FEAT], w1_ref[:D_FEAT, :],
                preferred_element_type=jnp.float32)
    h = h + jnp.dot(g[:, D_FEAT:], w1_ref[D_FEAT:, :],
                    preferred_element_type=jnp.float32) * (1.0 / S_NEIGH)
    h = jnp.maximum(h, 0.0)

    a = jnp.dot(h, wflat_ref[...],
                preferred_element_type=jnp.float32)

    rr = lax.broadcasted_iota(jnp.int32, (TC_ROWS, BB), 0)
    cc = lax.broadcasted_iota(jnp.int32, (TC_ROWS, BB), 1)
    psel = (rr == cc * M_PAD).astype(jnp.float32)
    pgrp = (rr // M_PAD == cc).astype(jnp.float32)

    def group_bcast(x):
        sel = jax.lax.dot_general(psel, x, (((0,), (0,)), ((), ())),
                                  preferred_element_type=jnp.float32)
        return jnp.dot(pgrp, sel, preferred_element_type=jnp.float32)

    abc = group_bcast(a)
    ls = jnp.sum((a * abc).reshape(BB, M_PAD, K_ASP * D_OUT), axis=-1)
    ls = ls * (1.0 / K_ASP)

    gmax = a[:, :D_OUT]
    for k in range(1, K_ASP):
        gmax = jnp.maximum(gmax, a[:, k * D_OUT:(k + 1) * D_OUT])
    gmaxbc = group_bcast(gmax)
    gs = jnp.sum((gmax * gmaxbc).reshape(BB, M_PAD, D_OUT), axis=-1)

    midx = lax.broadcasted_iota(jnp.int32, (BB, M_PAD), 1)
    valid = (midx >= 1) & (midx <= 1 + NUM_NEGS)

    def xent(scores):
        sm = jnp.where(valid, scores, -1e30)
        rmax = jnp.max(sm, axis=1, keepdims=True)
        se = jnp.sum(jnp.where(valid, jnp.exp(scores - rmax), 0.0),
                     axis=1, keepdims=True)
        row = jnp.log(se) + rmax - scores[:, 1:2]
        return jnp.sum(row) * (1.0 / BATCH)

    xent_g = xent(gs)
    xent_l = xent(ls)

    locs = [a[:, k * D_OUT:(k + 1) * D_OUT].reshape(BB, M_PAD, D_OUT)
            for k in range(K_ASP)]
    gram = [[None] * K_ASP for _ in range(K_ASP)]
    for k in range(K_ASP):
        for n in range(k, K_ASP):
            p = jnp.sum(locs[k] * locs[n], axis=-1)
            gram[k][n] = p
            gram[n][k] = p
    acc = jnp.zeros((BB, M_PAD), jnp.float32)
    for n in range(K_ASP):
        deno = gram[0][n]
        for k in range(1, K_ASP):
            deno = jnp.maximum(deno, gram[k][n])
        deno = jnp.where(deno == 0.0, 1.0, deno)
        inv = 1.0 / deno
        for k in range(K_ASP):
            tgt = 1.0 if k == n else 0.0
            acc = acc + jnp.abs(gram[k][n] * inv - tgt)
    w = jnp.where(midx == 0, 1.0 / BATCH,
                  jnp.where(valid, 1.0 / (BATCH * (1 + NUM_NEGS)), 0.0))
    constrain = jnp.sum(acc * w)

    contrib = ALPHA * xent_g + BETA * xent_l + GAMMA * constrain

    @pl.when(i == 0)
    def _():
        out_ref[...] = jnp.zeros((1, 1), jnp.float32)
    out_ref[...] = out_ref[...] + jnp.reshape(contrib, (1, 1))


def _tc_loss(G, W1, Wflat, interpret=False):
    return pl.pallas_call(
        _tc_loss_body,
        grid=(GRID,),
        in_specs=[
            pl.BlockSpec((TC_ROWS, 2 * D_FEAT), lambda i: (i, 0)),
            pl.BlockSpec((2 * D_FEAT, D_HIDDEN), lambda i: (0, 0)),
            pl.BlockSpec((D_HIDDEN, K_ASP * D_OUT), lambda i: (0, 0)),
        ],
        out_specs=pl.BlockSpec((1, 1), lambda i: (0, 0)),
        out_shape=jax.ShapeDtypeStruct((1, 1), jnp.float32),
        interpret=interpret,
    )(G, W1, Wflat)


def kernel(edges, negs, neigh_table, features, W1, W_asp):
    pad = jnp.zeros((BATCH, M_PAD - 2 - NUM_NEGS), jnp.int32)
    ids24 = jnp.concatenate(
        [edges[:, 0:1], edges[:, 1:2], negs, pad], axis=1)
    ids_flat = ids24.reshape(ROWS)
    Wflat = jnp.transpose(W_asp, (1, 0, 2)).reshape(D_HIDDEN, K_ASP * D_OUT)
    ntp = jnp.zeros((N_NODES, 128), jnp.int32).at[:, :S_NEIGH].set(neigh_table)
    gs = [_sc_gather(ids_flat[q * HALF_ROWS:(q + 1) * HALF_ROWS],
                     ntp, features) for q in range(2)]
    ls = [_tc_loss(g, W1, Wflat) for g in gs]
    return (ls[0] + ls[1])[0, 0]
